# Initial kernel scaffold; baseline (speedup 1.0000x reference)
import functools

import jax
import jax.numpy as jnp
from jax import lax
from jax.experimental import pallas as pl
from jax.experimental.pallas import tpu as pltpu
from jax.experimental.pallas import tpu_sc as plsc

N_NODES = 10000
FEAT = 128
N_GRAPHS = 64

NC = 2
NT = 16
NW = NC * NT
NP = 10016
SLICE = NP // NT

BLK = 1000
GRID = N_NODES // BLK


def _sc_mesh():
    return plsc.VectorSubcoreMesh(core_axis_name="c", subcore_axis_name="s")


def _deg_call(coli, ones_rows, zeros16):
    chunks = coli.shape[0]
    rpt = chunks // NW

    @functools.partial(
        pl.kernel,
        out_type=jax.ShapeDtypeStruct((NC, NP, 16), jnp.float32),
        mesh=_sc_mesh(),
        scratch_types=[
            pltpu.VMEM((rpt, 128), jnp.int32),
            pltpu.VMEM((128, 16), jnp.float32),
            pltpu.VMEM_SHARED((NP, 16), jnp.float32),
        ],
    )
    def deg_k(coli_hbm, ones_hbm, zeros_hbm, out_hbm, colv, onesv, deg_sh):
        c = lax.axis_index("c")
        s = lax.axis_index("s")
        wid = c * NT + s
        pltpu.sync_copy(zeros_hbm, deg_sh.at[pl.ds(s * SLICE, SLICE)])
        pltpu.sync_copy(coli_hbm.at[pl.ds(wid * rpt, rpt)], colv)
        pltpu.sync_copy(ones_hbm, onesv)
        plsc.subcore_barrier()

        def body(j, carry):
            pltpu.sync_copy(onesv, deg_sh.at[colv.at[j]], add=True)
            return carry

        lax.fori_loop(0, rpt, body, 0)
        plsc.subcore_barrier()
        pltpu.sync_copy(deg_sh.at[pl.ds(s * SLICE, SLICE)],
                        out_hbm.at[c, pl.ds(s * SLICE, SLICE)])

    return deg_k(coli, ones_rows, zeros16)


def _agg_call(hs, rowi, coli, zeros128):
    chunks = rowi.shape[0]
    rpt = chunks // NW

    @functools.partial(
        pl.kernel,
        out_type=jax.ShapeDtypeStruct((NC, NP, FEAT), jnp.float32),
        mesh=_sc_mesh(),
        scratch_types=[
            pltpu.VMEM((rpt, 128), jnp.int32),
            pltpu.VMEM((rpt, 128), jnp.int32),
            pltpu.VMEM((128, FEAT), jnp.float32),
            pltpu.VMEM_SHARED((NP, FEAT), jnp.float32),
            pltpu.SemaphoreType.DMA,
        ],
    )
    def agg_k(hs_hbm, rowi_hbm, coli_hbm, zeros_hbm, out_hbm,
              rowv, colv, gbuf, acc_sh, sem):
        c = lax.axis_index("c")
        s = lax.axis_index("s")
        wid = c * NT + s
        pltpu.sync_copy(zeros_hbm, acc_sh.at[pl.ds(s * SLICE, SLICE)])
        pltpu.sync_copy(rowi_hbm.at[pl.ds(wid * rpt, rpt)], rowv)
        pltpu.sync_copy(coli_hbm.at[pl.ds(wid * rpt, rpt)], colv)
        plsc.subcore_barrier()

        def body(j, carry):
            pltpu.async_copy(hs_hbm.at[rowv.at[j]], gbuf, sem).wait()
            pltpu.sync_copy(gbuf, acc_sh.at[colv.at[j]], add=True)
            return carry

        lax.fori_loop(0, rpt, body, 0)
        plsc.subcore_barrier()
        pltpu.sync_copy(acc_sh.at[pl.ds(s * SLICE, SLICE)],
                        out_hbm.at[c, pl.ds(s * SLICE, SLICE)])

    return agg_k(hs, rowi, coli, zeros128)


def _first_tc(degp, x, W1):
    def body(degp_ref, x_ref, w_ref, hs_ref, disb_ref):
        deg = degp_ref[0, :, 0:1] + degp_ref[1, :, 0:1] + 1.0
        dis = lax.rsqrt(deg)
        h = jnp.dot(x_ref[...], w_ref[...], preferred_element_type=jnp.float32)
        hs_ref[...] = h * dis
        disb_ref[...] = jnp.broadcast_to(dis, (BLK, FEAT))

    return pl.pallas_call(
        body,
        grid=(GRID,),
        in_specs=[
            pl.BlockSpec((NC, BLK, 16), lambda i: (0, i, 0)),
            pl.BlockSpec((BLK, FEAT), lambda i: (i, 0)),
            pl.BlockSpec((FEAT, FEAT), lambda i: (0, 0)),
        ],
        out_specs=[
            pl.BlockSpec((BLK, FEAT), lambda i: (i, 0)),
            pl.BlockSpec((BLK, FEAT), lambda i: (i, 0)),
        ],
        out_shape=[
            jax.ShapeDtypeStruct((N_NODES, FEAT), jnp.float32),
            jax.ShapeDtypeStruct((N_NODES, FEAT), jnp.float32),
        ],
    )(degp, x, W1)


def _mid_tc(accp, hs, disb, b, W):
    def body(accp_ref, hs_ref, disb_ref, b_ref, w_ref, out_ref):
        agg = accp_ref[0] + accp_ref[1] + hs_ref[...]
        xn = jnp.maximum(disb_ref[...] * agg + b_ref[...], 0.0)
        h = jnp.dot(xn, w_ref[...], preferred_element_type=jnp.float32)
        out_ref[...] = h * disb_ref[...]

    return pl.pallas_call(
        body,
        grid=(GRID,),
        in_specs=[
            pl.BlockSpec((NC, BLK, FEAT), lambda i: (0, i, 0)),
            pl.BlockSpec((BLK, FEAT), lambda i: (i, 0)),
            pl.BlockSpec((BLK, FEAT), lambda i: (i, 0)),
            pl.BlockSpec((1, FEAT), lambda i: (0, 0)),
            pl.BlockSpec((FEAT, FEAT), lambda i: (0, 0)),
        ],
        out_specs=pl.BlockSpec((BLK, FEAT), lambda i: (i, 0)),
        out_shape=jax.ShapeDtypeStruct((N_NODES, FEAT), jnp.float32),
    )(accp, hs, disb, b, W)


def _final_tc(accp, hs, disb, b, batch3, Wf1, bf1, Wf2, bf2):
    h3 = Wf1.shape[1]
    nout = Wf2.shape[1]

    def body(accp_ref, hs_ref, disb_ref, b_ref, batch_ref,
             wf1_ref, bf1_ref, wf2_ref, bf2_ref, out_ref, pooled, counts):
        i = pl.program_id(0)

        @pl.when(i == 0)
        def _():
            pooled[...] = jnp.zeros_like(pooled)
            counts[...] = jnp.zeros_like(counts)

        agg = accp_ref[0] + accp_ref[1] + hs_ref[...]
        x3 = jnp.maximum(disb_ref[...] * agg + b_ref[...], 0.0)
        batch_row = batch_ref[0]
        giota = lax.broadcasted_iota(jnp.int32, (N_GRAPHS, BLK), 0)
        onehot_t = (giota == batch_row).astype(jnp.float32)
        pooled[...] += lax.dot_general(
            onehot_t, x3, (((1,), (0,)), ((), ())),
            preferred_element_type=jnp.float32)
        counts[...] += jnp.broadcast_to(
            jnp.sum(onehot_t, axis=1, keepdims=True), (N_GRAPHS, FEAT))

        @pl.when(i == GRID - 1)
        def _():
            mean = pooled[...] / jnp.maximum(counts[...], 1.0)
            hmid = jnp.dot(mean, wf1_ref[...],
                           preferred_element_type=jnp.float32) + bf1_ref[...]
            out_ref[...] = jnp.dot(hmid, wf2_ref[...],
                                   preferred_element_type=jnp.float32) + bf2_ref[...]

    return pl.pallas_call(
        body,
        grid=(GRID,),
        in_specs=[
            pl.BlockSpec((NC, BLK, FEAT), lambda i: (0, i, 0)),
            pl.BlockSpec((BLK, FEAT), lambda i: (i, 0)),
            pl.BlockSpec((BLK, FEAT), lambda i: (i, 0)),
            pl.BlockSpec((1, FEAT), lambda i: (0, 0)),
            pl.BlockSpec((1, 1, BLK), lambda i: (i, 0, 0)),
            pl.BlockSpec((FEAT, h3), lambda i: (0, 0)),
            pl.BlockSpec((1, h3), lambda i: (0, 0)),
            pl.BlockSpec((h3, nout), lambda i: (0, 0)),
            pl.BlockSpec((1, nout), lambda i: (0, 0)),
        ],
        out_specs=pl.BlockSpec((N_GRAPHS, nout), lambda i: (0, 0)),
        out_shape=jax.ShapeDtypeStruct((N_GRAPHS, nout), jnp.float32),
        scratch_shapes=[
            pltpu.VMEM((N_GRAPHS, FEAT), jnp.float32),
            pltpu.VMEM((N_GRAPHS, FEAT), jnp.float32),
        ],
    )(accp, hs, disb, b, batch3, Wf1, bf1, Wf2, bf2)


def kernel(x, edge_index, batch, W1, b1, W2, b2, Wf1, bf1, Wf2, bf2):
    row = edge_index[0]
    col = edge_index[1]
    n_edges = row.shape[0]
    ep = ((n_edges + NW * 128 - 1) // (NW * 128)) * (NW * 128)
    pad = ep - n_edges
    rowp = jnp.concatenate(
        [row, jnp.zeros((pad,), jnp.int32)]).reshape(ep // 128, 128)
    colp = jnp.concatenate(
        [col, jnp.full((pad,), N_NODES, jnp.int32)]).reshape(ep // 128, 128)

    ones16 = jnp.zeros((128, 16), jnp.float32).at[:, 0].set(1.0)
    zeros16 = jnp.zeros((SLICE, 16), jnp.float32)
    zeros128 = jnp.zeros((SLICE, FEAT), jnp.float32)
    b1r = b1.reshape(1, FEAT)
    b2r = b2.reshape(1, FEAT)
    bf1r = bf1.reshape(1, -1)
    bf2r = bf2.reshape(1, -1)
    batch3 = batch.reshape(GRID, 1, BLK)

    degp = _deg_call(colp, ones16, zeros16)
    hs1, disb = _first_tc(degp, x, W1)
    a1 = _agg_call(hs1, rowp, colp, zeros128)
    hs2 = _mid_tc(a1, hs1, disb, b1r, W2)
    a2 = _agg_call(hs2, rowp, colp, zeros128)
    return _final_tc(a2, hs2, disb, b2r, batch3, Wf1, bf1r, Wf2, bf2r)


# speedup vs baseline: 20.0369x; 20.0369x over previous
import functools

import jax
import jax.numpy as jnp
from jax import lax
from jax.experimental import pallas as pl
from jax.experimental.pallas import tpu as pltpu
from jax.experimental.pallas import tpu_sc as plsc

N_NODES = 10000
FEAT = 128
N_GRAPHS = 64

NC = 2
NT = 16
NW = NC * NT
NP = 10112
SLICE = NP // NT

BLK = 1000
GRID = N_NODES // BLK


def _sc_mesh():
    return plsc.VectorSubcoreMesh(core_axis_name="c", subcore_axis_name="s")


def _deg_call(coli, ones128, zeros128):
    chunks = coli.shape[0]
    rpt = chunks // NW

    @functools.partial(
        pl.kernel,
        out_type=jax.ShapeDtypeStruct((NC, NP, FEAT), jnp.float32),
        mesh=_sc_mesh(),
        scratch_types=[
            pltpu.VMEM((rpt, 128), jnp.int32),
            pltpu.VMEM((128, FEAT), jnp.float32),
            pltpu.VMEM_SHARED((NP, FEAT), jnp.float32),
        ],
    )
    def deg_k(coli_hbm, ones_hbm, zeros_hbm, out_hbm, colv, onesv, deg_sh):
        c = lax.axis_index("c")
        s = lax.axis_index("s")
        wid = c * NT + s
        pltpu.sync_copy(zeros_hbm, deg_sh.at[pl.ds(s * SLICE, SLICE)])
        pltpu.sync_copy(coli_hbm.at[pl.ds(wid * rpt, rpt)], colv)
        pltpu.sync_copy(ones_hbm, onesv)
        plsc.subcore_barrier()

        def body(j, carry):
            pltpu.sync_copy(onesv, deg_sh.at[colv.at[j]], add=True)
            return carry

        lax.fori_loop(0, rpt, body, 0)
        plsc.subcore_barrier()
        pltpu.sync_copy(deg_sh.at[pl.ds(s * SLICE, SLICE)],
                        out_hbm.at[c, pl.ds(s * SLICE, SLICE)])

    return deg_k(coli, ones128, zeros128)


def _agg_call(hs, rowi, coli, zeros128):
    chunks = rowi.shape[0]
    rpt = chunks // NW

    @functools.partial(
        pl.kernel,
        out_type=jax.ShapeDtypeStruct((NC, NP, FEAT), jnp.float32),
        mesh=_sc_mesh(),
        scratch_types=[
            pltpu.VMEM((rpt, 128), jnp.int32),
            pltpu.VMEM((rpt, 128), jnp.int32),
            pltpu.VMEM((128, FEAT), jnp.float32),
            pltpu.VMEM_SHARED((NP, FEAT), jnp.float32),
            pltpu.SemaphoreType.DMA,
        ],
    )
    def agg_k(hs_hbm, rowi_hbm, coli_hbm, zeros_hbm, out_hbm,
              rowv, colv, gbuf, acc_sh, sem):
        c = lax.axis_index("c")
        s = lax.axis_index("s")
        wid = c * NT + s
        pltpu.sync_copy(zeros_hbm, acc_sh.at[pl.ds(s * SLICE, SLICE)])
        pltpu.sync_copy(rowi_hbm.at[pl.ds(wid * rpt, rpt)], rowv)
        pltpu.sync_copy(coli_hbm.at[pl.ds(wid * rpt, rpt)], colv)
        plsc.subcore_barrier()

        def body(j, carry):
            pltpu.async_copy(hs_hbm.at[rowv.at[j]], gbuf, sem).wait()
            pltpu.sync_copy(gbuf, acc_sh.at[colv.at[j]], add=True)
            return carry

        lax.fori_loop(0, rpt, body, 0)
        plsc.subcore_barrier()
        pltpu.sync_copy(acc_sh.at[pl.ds(s * SLICE, SLICE)],
                        out_hbm.at[c, pl.ds(s * SLICE, SLICE)])

    return agg_k(hs, rowi, coli, zeros128)


def _first_tc(degp, x, W1):
    def body(degp_ref, x_ref, w_ref, hs_ref, disb_ref):
        deg = degp_ref[0, :, 0:1] + degp_ref[1, :, 0:1] + 1.0
        dis = lax.rsqrt(deg)
        h = jnp.dot(x_ref[...], w_ref[...], preferred_element_type=jnp.float32)
        hs_ref[...] = h * dis
        disb_ref[...] = jnp.broadcast_to(dis, (BLK, FEAT))

    return pl.pallas_call(
        body,
        grid=(GRID,),
        in_specs=[
            pl.BlockSpec((NC, BLK, FEAT), lambda i: (0, i, 0)),
            pl.BlockSpec((BLK, FEAT), lambda i: (i, 0)),
            pl.BlockSpec((FEAT, FEAT), lambda i: (0, 0)),
        ],
        out_specs=[
            pl.BlockSpec((BLK, FEAT), lambda i: (i, 0)),
            pl.BlockSpec((BLK, FEAT), lambda i: (i, 0)),
        ],
        out_shape=[
            jax.ShapeDtypeStruct((N_NODES, FEAT), jnp.float32),
            jax.ShapeDtypeStruct((N_NODES, FEAT), jnp.float32),
        ],
    )(degp, x, W1)


def _mid_tc(accp, hs, disb, b, W):
    def body(accp_ref, hs_ref, disb_ref, b_ref, w_ref, out_ref):
        agg = accp_ref[0] + accp_ref[1] + hs_ref[...]
        xn = jnp.maximum(disb_ref[...] * agg + b_ref[...], 0.0)
        h = jnp.dot(xn, w_ref[...], preferred_element_type=jnp.float32)
        out_ref[...] = h * disb_ref[...]

    return pl.pallas_call(
        body,
        grid=(GRID,),
        in_specs=[
            pl.BlockSpec((NC, BLK, FEAT), lambda i: (0, i, 0)),
            pl.BlockSpec((BLK, FEAT), lambda i: (i, 0)),
            pl.BlockSpec((BLK, FEAT), lambda i: (i, 0)),
            pl.BlockSpec((1, FEAT), lambda i: (0, 0)),
            pl.BlockSpec((FEAT, FEAT), lambda i: (0, 0)),
        ],
        out_specs=pl.BlockSpec((BLK, FEAT), lambda i: (i, 0)),
        out_shape=jax.ShapeDtypeStruct((N_NODES, FEAT), jnp.float32),
    )(accp, hs, disb, b, W)


def _final_tc(accp, hs, disb, b, batch3, Wf1, bf1, Wf2, bf2):
    h3 = Wf1.shape[1]
    nout = Wf2.shape[1]

    def body(accp_ref, hs_ref, disb_ref, b_ref, batch_ref,
             wf1_ref, bf1_ref, wf2_ref, bf2_ref, out_ref, pooled, counts):
        i = pl.program_id(0)

        @pl.when(i == 0)
        def _():
            pooled[...] = jnp.zeros_like(pooled)
            counts[...] = jnp.zeros_like(counts)

        agg = accp_ref[0] + accp_ref[1] + hs_ref[...]
        x3 = jnp.maximum(disb_ref[...] * agg + b_ref[...], 0.0)
        batch_row = batch_ref[0]
        giota = lax.broadcasted_iota(jnp.int32, (N_GRAPHS, BLK), 0)
        onehot_t = (giota == batch_row).astype(jnp.float32)
        pooled[...] += lax.dot_general(
            onehot_t, x3, (((1,), (0,)), ((), ())),
            preferred_element_type=jnp.float32)
        counts[...] += jnp.broadcast_to(
            jnp.sum(onehot_t, axis=1, keepdims=True), (N_GRAPHS, FEAT))

        @pl.when(i == GRID - 1)
        def _():
            mean = pooled[...] / jnp.maximum(counts[...], 1.0)
            hmid = jnp.dot(mean, wf1_ref[...],
                           preferred_element_type=jnp.float32) + bf1_ref[...]
            out_ref[...] = jnp.dot(hmid, wf2_ref[...],
                                   preferred_element_type=jnp.float32) + bf2_ref[...]

    return pl.pallas_call(
        body,
        grid=(GRID,),
        in_specs=[
            pl.BlockSpec((NC, BLK, FEAT), lambda i: (0, i, 0)),
            pl.BlockSpec((BLK, FEAT), lambda i: (i, 0)),
            pl.BlockSpec((BLK, FEAT), lambda i: (i, 0)),
            pl.BlockSpec((1, FEAT), lambda i: (0, 0)),
            pl.BlockSpec((1, 1, BLK), lambda i: (i, 0, 0)),
            pl.BlockSpec((FEAT, h3), lambda i: (0, 0)),
            pl.BlockSpec((1, h3), lambda i: (0, 0)),
            pl.BlockSpec((h3, nout), lambda i: (0, 0)),
            pl.BlockSpec((1, nout), lambda i: (0, 0)),
        ],
        out_specs=pl.BlockSpec((N_GRAPHS, nout), lambda i: (0, 0)),
        out_shape=jax.ShapeDtypeStruct((N_GRAPHS, nout), jnp.float32),
        scratch_shapes=[
            pltpu.VMEM((N_GRAPHS, FEAT), jnp.float32),
            pltpu.VMEM((N_GRAPHS, FEAT), jnp.float32),
        ],
    )(accp, hs, disb, b, batch3, Wf1, bf1, Wf2, bf2)


def kernel(x, edge_index, batch, W1, b1, W2, b2, Wf1, bf1, Wf2, bf2):
    row = edge_index[0]
    col = edge_index[1]
    n_edges = row.shape[0]
    quantum = NW * 128 * 8
    ep = ((n_edges + quantum - 1) // quantum) * quantum
    pad = ep - n_edges
    pad_iota = jnp.arange(pad, dtype=jnp.int32)
    rowp = jnp.concatenate(
        [row, pad_iota % N_NODES]).reshape(ep // 128, 128)
    colp = jnp.concatenate(
        [col, N_NODES + pad_iota % (NP - N_NODES)]).reshape(ep // 128, 128)

    ones128 = jnp.ones((128, FEAT), jnp.float32)
    zeros128 = jnp.zeros((SLICE, FEAT), jnp.float32)
    b1r = b1.reshape(1, FEAT)
    b2r = b2.reshape(1, FEAT)
    bf1r = bf1.reshape(1, -1)
    bf2r = bf2.reshape(1, -1)
    batch3 = batch.reshape(GRID, 1, BLK)

    degp = _deg_call(colp, ones128, zeros128)
    hs1, disb = _first_tc(degp, x, W1)
    a1 = _agg_call(hs1, rowp, colp, zeros128)
    hs2 = _mid_tc(a1, hs1, disb, b1r, W2)
    a2 = _agg_call(hs2, rowp, colp, zeros128)
    return _final_tc(a2, hs2, disb, b2r, batch3, Wf1, bf1r, Wf2, bf2r)


# speedup vs baseline: 23.8085x; 1.1882x over previous
import functools

import jax
import jax.numpy as jnp
from jax import lax
from jax.experimental import pallas as pl
from jax.experimental.pallas import tpu as pltpu
from jax.experimental.pallas import tpu_sc as plsc

N_NODES = 10000
FEAT = 128
N_GRAPHS = 64

NC = 2
NT = 16
NW = NC * NT
NP = 10112
SLICE = NP // NT

BLK = 1000
GRID = N_NODES // BLK


def _sc_mesh():
    return plsc.VectorSubcoreMesh(core_axis_name="c", subcore_axis_name="s")


def _deg_call(coli, ones128, zeros128):
    chunks = coli.shape[0]
    rpt = chunks // NW

    @functools.partial(
        pl.kernel,
        out_type=jax.ShapeDtypeStruct((NC, NP, FEAT), jnp.float32),
        mesh=_sc_mesh(),
        scratch_types=[
            pltpu.VMEM((rpt, 128), jnp.int32),
            pltpu.VMEM((128, FEAT), jnp.float32),
            pltpu.VMEM_SHARED((NP, FEAT), jnp.float32),
        ],
    )
    def deg_k(coli_hbm, ones_hbm, zeros_hbm, out_hbm, colv, onesv, deg_sh):
        c = lax.axis_index("c")
        s = lax.axis_index("s")
        wid = c * NT + s
        pltpu.sync_copy(zeros_hbm, deg_sh.at[pl.ds(s * SLICE, SLICE)])
        pltpu.sync_copy(coli_hbm.at[pl.ds(wid * rpt, rpt)], colv)
        pltpu.sync_copy(ones_hbm, onesv)
        plsc.subcore_barrier()

        def body(j, carry):
            pltpu.sync_copy(onesv, deg_sh.at[colv.at[j]], add=True)
            return carry

        lax.fori_loop(0, rpt, body, 0)
        plsc.subcore_barrier()
        pltpu.sync_copy(deg_sh.at[pl.ds(s * SLICE, SLICE)],
                        out_hbm.at[c, pl.ds(s * SLICE, SLICE)])

    return deg_k(coli, ones128, zeros128)


def _agg_call(hs, rowi, coli, zeros128):
    chunks = rowi.shape[0]
    rpt = chunks // NW

    @functools.partial(
        pl.kernel,
        out_type=jax.ShapeDtypeStruct((NC, NP, FEAT), jnp.float32),
        mesh=_sc_mesh(),
        scratch_types=[
            pltpu.VMEM((16, 128), jnp.int32),
            pltpu.VMEM((16, 128), jnp.int32),
            pltpu.VMEM((128, FEAT), jnp.float32),
            pltpu.VMEM((128, FEAT), jnp.float32),
            pltpu.VMEM_SHARED((NP, FEAT), jnp.float32),
            pltpu.SemaphoreType.DMA,
        ],
    )
    def agg_k(hs_hbm, rowi_hbm, coli_hbm, zeros_hbm, out_hbm,
              rowv, colv, gbuf0, gbuf1, acc_sh, sem):
        c = lax.axis_index("c")
        s = lax.axis_index("s")
        wid = c * NT + s
        seg = 16
        nseg = rpt // seg
        pltpu.sync_copy(zeros_hbm, acc_sh.at[pl.ds(s * SLICE, SLICE)])
        plsc.subcore_barrier()

        def seg_body(g, carry):
            base = wid * rpt + g * seg
            pltpu.sync_copy(rowi_hbm.at[pl.ds(base, seg)], rowv)
            pltpu.sync_copy(coli_hbm.at[pl.ds(base, seg)], colv)
            pltpu.async_copy(hs_hbm.at[rowv.at[0]], gbuf0, sem)

            def body(t, c2):
                j0 = t * 2
                pltpu.make_async_copy(hs_hbm.at[rowv.at[0]], gbuf0, sem).wait()
                pltpu.async_copy(hs_hbm.at[rowv.at[j0 + 1]], gbuf1, sem)
                pltpu.sync_copy(gbuf0, acc_sh.at[colv.at[j0]], add=True)
                pltpu.make_async_copy(hs_hbm.at[rowv.at[0]], gbuf1, sem).wait()

                @pl.when(t + 1 < seg // 2)
                def _():
                    pltpu.async_copy(hs_hbm.at[rowv.at[j0 + 2]], gbuf0, sem)

                pltpu.sync_copy(gbuf1, acc_sh.at[colv.at[j0 + 1]], add=True)
                return c2

            lax.fori_loop(0, seg // 2, body, 0)
            return carry

        lax.fori_loop(0, nseg, seg_body, 0)
        plsc.subcore_barrier()
        pltpu.sync_copy(acc_sh.at[pl.ds(s * SLICE, SLICE)],
                        out_hbm.at[c, pl.ds(s * SLICE, SLICE)])

    return agg_k(hs, rowi, coli, zeros128)


def _first_tc(degp, x, W1):
    def body(degp_ref, x_ref, w_ref, hs_ref, disb_ref):
        deg = degp_ref[0, :, 0:1] + degp_ref[1, :, 0:1] + 1.0
        dis = lax.rsqrt(deg)
        h = jnp.dot(x_ref[...], w_ref[...], preferred_element_type=jnp.float32)
        hs_ref[...] = h * dis
        disb_ref[...] = jnp.broadcast_to(dis, (BLK, FEAT))

    return pl.pallas_call(
        body,
        grid=(GRID,),
        in_specs=[
            pl.BlockSpec((NC, BLK, FEAT), lambda i: (0, i, 0)),
            pl.BlockSpec((BLK, FEAT), lambda i: (i, 0)),
            pl.BlockSpec((FEAT, FEAT), lambda i: (0, 0)),
        ],
        out_specs=[
            pl.BlockSpec((BLK, FEAT), lambda i: (i, 0)),
            pl.BlockSpec((BLK, FEAT), lambda i: (i, 0)),
        ],
        out_shape=[
            jax.ShapeDtypeStruct((N_NODES, FEAT), jnp.float32),
            jax.ShapeDtypeStruct((N_NODES, FEAT), jnp.float32),
        ],
    )(degp, x, W1)


def _mid_tc(accp, hs, disb, b, W):
    def body(accp_ref, hs_ref, disb_ref, b_ref, w_ref, out_ref):
        agg = accp_ref[0] + accp_ref[1] + hs_ref[...]
        xn = jnp.maximum(disb_ref[...] * agg + b_ref[...], 0.0)
        h = jnp.dot(xn, w_ref[...], preferred_element_type=jnp.float32)
        out_ref[...] = h * disb_ref[...]

    return pl.pallas_call(
        body,
        grid=(GRID,),
        in_specs=[
            pl.BlockSpec((NC, BLK, FEAT), lambda i: (0, i, 0)),
            pl.BlockSpec((BLK, FEAT), lambda i: (i, 0)),
            pl.BlockSpec((BLK, FEAT), lambda i: (i, 0)),
            pl.BlockSpec((1, FEAT), lambda i: (0, 0)),
            pl.BlockSpec((FEAT, FEAT), lambda i: (0, 0)),
        ],
        out_specs=pl.BlockSpec((BLK, FEAT), lambda i: (i, 0)),
        out_shape=jax.ShapeDtypeStruct((N_NODES, FEAT), jnp.float32),
    )(accp, hs, disb, b, W)


def _final_tc(accp, hs, disb, b, batch3, Wf1, bf1, Wf2, bf2):
    h3 = Wf1.shape[1]
    nout = Wf2.shape[1]

    def body(accp_ref, hs_ref, disb_ref, b_ref, batch_ref,
             wf1_ref, bf1_ref, wf2_ref, bf2_ref, out_ref, pooled, counts):
        i = pl.program_id(0)

        @pl.when(i == 0)
        def _():
            pooled[...] = jnp.zeros_like(pooled)
            counts[...] = jnp.zeros_like(counts)

        agg = accp_ref[0] + accp_ref[1] + hs_ref[...]
        x3 = jnp.maximum(disb_ref[...] * agg + b_ref[...], 0.0)
        batch_row = batch_ref[0]
        giota = lax.broadcasted_iota(jnp.int32, (N_GRAPHS, BLK), 0)
        onehot_t = (giota == batch_row).astype(jnp.float32)
        pooled[...] += lax.dot_general(
            onehot_t, x3, (((1,), (0,)), ((), ())),
            preferred_element_type=jnp.float32)
        counts[...] += jnp.broadcast_to(
            jnp.sum(onehot_t, axis=1, keepdims=True), (N_GRAPHS, FEAT))

        @pl.when(i == GRID - 1)
        def _():
            mean = pooled[...] / jnp.maximum(counts[...], 1.0)
            hmid = jnp.dot(mean, wf1_ref[...],
                           preferred_element_type=jnp.float32) + bf1_ref[...]
            out_ref[...] = jnp.dot(hmid, wf2_ref[...],
                                   preferred_element_type=jnp.float32) + bf2_ref[...]

    return pl.pallas_call(
        body,
        grid=(GRID,),
        in_specs=[
            pl.BlockSpec((NC, BLK, FEAT), lambda i: (0, i, 0)),
            pl.BlockSpec((BLK, FEAT), lambda i: (i, 0)),
            pl.BlockSpec((BLK, FEAT), lambda i: (i, 0)),
            pl.BlockSpec((1, FEAT), lambda i: (0, 0)),
            pl.BlockSpec((1, 1, BLK), lambda i: (i, 0, 0)),
            pl.BlockSpec((FEAT, h3), lambda i: (0, 0)),
            pl.BlockSpec((1, h3), lambda i: (0, 0)),
            pl.BlockSpec((h3, nout), lambda i: (0, 0)),
            pl.BlockSpec((1, nout), lambda i: (0, 0)),
        ],
        out_specs=pl.BlockSpec((N_GRAPHS, nout), lambda i: (0, 0)),
        out_shape=jax.ShapeDtypeStruct((N_GRAPHS, nout), jnp.float32),
        scratch_shapes=[
            pltpu.VMEM((N_GRAPHS, FEAT), jnp.float32),
            pltpu.VMEM((N_GRAPHS, FEAT), jnp.float32),
        ],
    )(accp, hs, disb, b, batch3, Wf1, bf1, Wf2, bf2)


def kernel(x, edge_index, batch, W1, b1, W2, b2, Wf1, bf1, Wf2, bf2):
    row = edge_index[0]
    col = edge_index[1]
    n_edges = row.shape[0]
    quantum = NW * 128 * 8
    ep = ((n_edges + quantum - 1) // quantum) * quantum
    pad = ep - n_edges
    pad_iota = jnp.arange(pad, dtype=jnp.int32)
    rowp = jnp.concatenate(
        [row, pad_iota % N_NODES]).reshape(ep // 128, 128)
    colp = jnp.concatenate(
        [col, N_NODES + pad_iota % (NP - N_NODES)]).reshape(ep // 128, 128)

    ones128 = jnp.ones((128, FEAT), jnp.float32)
    zeros128 = jnp.zeros((SLICE, FEAT), jnp.float32)
    b1r = b1.reshape(1, FEAT)
    b2r = b2.reshape(1, FEAT)
    bf1r = bf1.reshape(1, -1)
    bf2r = bf2.reshape(1, -1)
    batch3 = batch.reshape(GRID, 1, BLK)

    degp = _deg_call(colp, ones128, zeros128)
    hs1, disb = _first_tc(degp, x, W1)
    a1 = _agg_call(hs1, rowp, colp, zeros128)
    hs2 = _mid_tc(a1, hs1, disb, b1r, W2)
    a2 = _agg_call(hs2, rowp, colp, zeros128)
    return _final_tc(a2, hs2, disb, b2r, batch3, Wf1, bf1r, Wf2, bf2r)
